# baseline (speedup 1.0000x reference)
import jax
import jax.numpy as jnp
from jax import lax
from jax.experimental import pallas as pl
from jax.experimental.pallas import tpu as pltpu
from jax.experimental.pallas import tpu_sc as plsc

_M = 4096
_N = 16384
_NTEC = 32
_QPT = _M // _NTEC
_G = _QPT // 16
_GP = 4


def _splat_f32(v):
    return jnp.full((16,), 0, dtype=jnp.float32) + v


def _splat_i32(v):
    return jnp.full((16,), 0, dtype=jnp.int32) + v


def _sc_body(ax_hbm, ay_hbm, bx_hbm, by_hbm, dist_hbm, idx_hbm,
             ax_v, ay_v, bx_v, by_v, dist_v, idx_v):
    cid = lax.axis_index("c")
    sid = lax.axis_index("s")
    wid = sid * 2 + cid
    qbase = wid * _QPT

    pltpu.sync_copy(ax_hbm, ax_v)
    pltpu.sync_copy(ay_hbm, ay_v)
    pltpu.sync_copy(bx_hbm.at[pl.ds(qbase, _QPT)], bx_v)
    pltpu.sync_copy(by_hbm.at[pl.ds(qbase, _QPT)], by_v)

    inf_f = jnp.full((16,), jnp.float32(jnp.inf))
    zero_i = jnp.zeros((16,), jnp.int32)

    for p in range(_G // _GP):
        qx = [bx_v[pl.ds((p * _GP + i) * 16, 16)] for i in range(_GP)]
        qy = [by_v[pl.ds((p * _GP + i) * 16, 16)] for i in range(_GP)]

        def pbody(blk, carry, qx=qx, qy=qy):
            rmins, ridxs = carry
            ax16 = ax_v[pl.ds(blk * 16, 16)]
            ay16 = ay_v[pl.ds(blk * 16, 16)]
            rmins = list(rmins)
            ridxs = list(ridxs)
            for t in range(16):
                axb = _splat_f32(ax16[t])
                ayb = _splat_f32(ay16[t])
                jv = _splat_i32(blk * 16 + t)
                for i in range(_GP):
                    dx = qx[i] - axb
                    dy = qy[i] - ayb
                    d2 = dx * dx + dy * dy
                    m = d2 < rmins[i]
                    rmins[i] = jnp.where(m, d2, rmins[i])
                    ridxs[i] = jnp.where(m, jv, ridxs[i])
            return tuple(rmins), tuple(ridxs)

        rmins, ridxs = lax.fori_loop(
            0, _N // 16, pbody, ((inf_f,) * _GP, (zero_i,) * _GP))

        for i in range(_GP):
            x = rmins[i]
            bi = lax.bitcast_convert_type(x, jnp.int32)
            bi = jnp.int32(0x5F3759DF) - lax.shift_right_logical(bi, 1)
            y = lax.bitcast_convert_type(bi, jnp.float32)
            for _ in range(3):
                y = y * (1.5 - 0.5 * x * y * y)
            dist_v[pl.ds((p * _GP + i) * 16, 16)] = x * y
            idx_v[pl.ds((p * _GP + i) * 16, 16)] = ridxs[i]

    pltpu.sync_copy(dist_v, dist_hbm.at[pl.ds(qbase, _QPT)])
    pltpu.sync_copy(idx_v, idx_hbm.at[pl.ds(qbase, _QPT)])


def _make_sc_nn():
    mesh = plsc.VectorSubcoreMesh(
        core_axis_name="c", subcore_axis_name="s",
        num_cores=2, num_subcores=16)
    return pl.kernel(
        _sc_body,
        out_type=(jax.ShapeDtypeStruct((_M,), jnp.float32),
                  jax.ShapeDtypeStruct((_M,), jnp.int32)),
        mesh=mesh,
        scratch_types=[
            pltpu.VMEM((_N,), jnp.float32),
            pltpu.VMEM((_N,), jnp.float32),
            pltpu.VMEM((_QPT,), jnp.float32),
            pltpu.VMEM((_QPT,), jnp.float32),
            pltpu.VMEM((_QPT,), jnp.float32),
            pltpu.VMEM((_QPT,), jnp.int32),
        ],
    )


def kernel(A, B, k):
    ax = A[:, 0]
    ay = A[:, 1]
    bx = B[:, 0]
    by = B[:, 1]
    dist, idx = _make_sc_nn()(ax, ay, bx, by)
    return dist.reshape(_M, 1), (idx * jnp.asarray(k, idx.dtype)).reshape(_M, 1)


# speedup vs baseline: 5.0065x; 5.0065x over previous
import jax
import jax.numpy as jnp
from jax import lax
from jax.experimental import pallas as pl
from jax.experimental.pallas import tpu as pltpu
from jax.experimental.pallas import tpu_sc as plsc

_M = 4096
_N = 16384
_NTEC = 32

_MS = 1024
_QPT = _MS // _NTEC
_G = _QPT // 16
_GP = min(_G, 4)

_MT = _M - _MS
_QT = 256
_NC = 2048


def _splat_f32(v):
    return jnp.full((16,), 0, dtype=jnp.float32) + v


def _splat_i32(v):
    return jnp.full((16,), 0, dtype=jnp.int32) + v


def _sc_body(ax_hbm, ay_hbm, bx_hbm, by_hbm, dist_hbm, idx_hbm,
             ax_v, ay_v, bx_v, by_v, dist_v, idx_v):
    cid = lax.axis_index("c")
    sid = lax.axis_index("s")
    wid = sid * 2 + cid
    qbase = wid * _QPT

    pltpu.sync_copy(ax_hbm, ax_v)
    pltpu.sync_copy(ay_hbm, ay_v)
    pltpu.sync_copy(bx_hbm.at[pl.ds(qbase, _QPT)], bx_v)
    pltpu.sync_copy(by_hbm.at[pl.ds(qbase, _QPT)], by_v)

    inf_f = jnp.full((16,), jnp.float32(jnp.inf))
    zero_i = jnp.zeros((16,), jnp.int32)

    for p in range(_G // _GP):
        qx = [bx_v[pl.ds((p * _GP + i) * 16, 16)] for i in range(_GP)]
        qy = [by_v[pl.ds((p * _GP + i) * 16, 16)] for i in range(_GP)]

        def pbody(blk, carry, qx=qx, qy=qy):
            rmins, ridxs = carry
            ax16 = ax_v[pl.ds(blk * 16, 16)]
            ay16 = ay_v[pl.ds(blk * 16, 16)]
            rmins = list(rmins)
            ridxs = list(ridxs)
            for t in range(16):
                axb = _splat_f32(ax16[t])
                ayb = _splat_f32(ay16[t])
                jv = _splat_i32(blk * 16 + t)
                for i in range(_GP):
                    dx = qx[i] - axb
                    dy = qy[i] - ayb
                    d2 = dx * dx + dy * dy
                    m = d2 < rmins[i]
                    rmins[i] = jnp.where(m, d2, rmins[i])
                    ridxs[i] = jnp.where(m, jv, ridxs[i])
            return tuple(rmins), tuple(ridxs)

        rmins, ridxs = lax.fori_loop(
            0, _N // 16, pbody, ((inf_f,) * _GP, (zero_i,) * _GP))

        for i in range(_GP):
            x = rmins[i]
            bi = lax.bitcast_convert_type(x, jnp.int32)
            bi = jnp.int32(0x5F3759DF) - lax.shift_right_logical(bi, 1)
            y = lax.bitcast_convert_type(bi, jnp.float32)
            for _ in range(3):
                y = y * (1.5 - 0.5 * x * y * y)
            dist_v[pl.ds((p * _GP + i) * 16, 16)] = x * y
            idx_v[pl.ds((p * _GP + i) * 16, 16)] = ridxs[i]

    pltpu.sync_copy(dist_v, dist_hbm.at[pl.ds(qbase, _QPT)])
    pltpu.sync_copy(idx_v, idx_hbm.at[pl.ds(qbase, _QPT)])


def _make_sc_nn():
    mesh = plsc.VectorSubcoreMesh(
        core_axis_name="c", subcore_axis_name="s",
        num_cores=2, num_subcores=16)
    return pl.kernel(
        _sc_body,
        out_type=(jax.ShapeDtypeStruct((_MS,), jnp.float32),
                  jax.ShapeDtypeStruct((_MS,), jnp.int32)),
        mesh=mesh,
        scratch_types=[
            pltpu.VMEM((_N,), jnp.float32),
            pltpu.VMEM((_N,), jnp.float32),
            pltpu.VMEM((_QPT,), jnp.float32),
            pltpu.VMEM((_QPT,), jnp.float32),
            pltpu.VMEM((_QPT,), jnp.float32),
            pltpu.VMEM((_QPT,), jnp.int32),
        ],
    )


def _tc_body(b_ref, at_ref, dist_ref, idx_ref):
    qx = b_ref[:, 0:1]
    qy = b_ref[:, 1:2]

    def body(j, carry):
        rmin, ridx = carry
        ax = at_ref[0:1, pl.ds(j * _NC, _NC)]
        ay = at_ref[1:2, pl.ds(j * _NC, _NC)]
        dx = qx - ax
        dy = qy - ay
        d2 = dx * dx + dy * dy
        cmin = jnp.min(d2, axis=1, keepdims=True)
        iota = jax.lax.broadcasted_iota(jnp.int32, (_QT, _NC), 1) + j * _NC
        cidx = jnp.min(
            jnp.where(d2 == cmin, iota, jnp.int32(2**30)),
            axis=1, keepdims=True)
        upd = cmin < rmin
        return jnp.where(upd, cmin, rmin), jnp.where(upd, cidx, ridx)

    rmin0 = jnp.full((_QT, 1), jnp.inf, jnp.float32)
    ridx0 = jnp.zeros((_QT, 1), jnp.int32)
    rmin, ridx = jax.lax.fori_loop(0, _N // _NC, body, (rmin0, ridx0))
    dist_ref[:, :] = jnp.sqrt(rmin)
    idx_ref[:, :] = ridx


def _tc_nn(b_tail, at):
    return pl.pallas_call(
        _tc_body,
        grid=(_MT // _QT,),
        in_specs=[
            pl.BlockSpec((_QT, 2), lambda i: (i, 0)),
            pl.BlockSpec((2, _N), lambda i: (0, 0)),
        ],
        out_specs=[
            pl.BlockSpec((_QT, 1), lambda i: (i, 0)),
            pl.BlockSpec((_QT, 1), lambda i: (i, 0)),
        ],
        out_shape=[
            jax.ShapeDtypeStruct((_MT, 1), jnp.float32),
            jax.ShapeDtypeStruct((_MT, 1), jnp.int32),
        ],
    )(b_tail, at)


def kernel(A, B, k):
    ax = A[:, 0]
    ay = A[:, 1]
    bx = B[:_MS, 0]
    by = B[:_MS, 1]
    dist_sc, idx_sc = _make_sc_nn()(ax, ay, bx, by)
    at = A.T
    dist_tc, idx_tc = _tc_nn(B[_MS:], at)
    dist = jnp.concatenate([dist_sc.reshape(_MS, 1), dist_tc], axis=0)
    idx = jnp.concatenate([idx_sc.reshape(_MS, 1), idx_tc], axis=0)
    return dist, idx * jnp.asarray(k, idx.dtype)
